# baseline (speedup 1.0000x reference)
import jax
import jax.numpy as jnp
from jax import lax
from jax.experimental import pallas as pl
from jax.experimental.pallas import tpu as pltpu
from jax.experimental.pallas import tpu_sc as plsc

B = 4096
L = 200
D = 64
N = B * L
NC = 2
NS = 16
NW = NC * NS
PER_W = N // NW
C = 512
N_CHUNKS = PER_W // C
LANES = 16


def _body(year_hbm, month_hbm, pos_hbm, x_hbm, out_hbm,
          xch_v, iy_v, im_v, ip_v, acc_v, b1_v, b2_v, sem):
    cid = lax.axis_index("c")
    sid = lax.axis_index("s")
    wid = sid * NC + cid
    base = wid * PER_W
    lane = lax.iota(jnp.int32, LANES)

    def chunk_body(k, carry):
        off = pl.multiple_of(base + k * C, C)
        pltpu.sync_copy(x_hbm.at[pl.ds(off * 3, C * 3)], xch_v)

        def deint_body(g, gcarry):
            rowv = (g * LANES + lane) * 3
            iy_v[pl.ds(g * LANES, LANES)] = plsc.load_gather(xch_v, [rowv])
            im_v[pl.ds(g * LANES, LANES)] = plsc.load_gather(xch_v, [rowv + 1])
            ip_v[pl.ds(g * LANES, LANES)] = plsc.load_gather(xch_v, [rowv + 2])
            return gcarry

        lax.fori_loop(0, C // LANES, deint_body, 0, unroll=False)

        cp0 = pltpu.async_copy(year_hbm.at[iy_v], acc_v, sem)
        cp1 = pltpu.async_copy(month_hbm.at[im_v], b1_v, sem)
        cp2 = pltpu.async_copy(pos_hbm.at[ip_v], b2_v, sem)
        cp0.wait()
        cp1.wait()
        cp2.wait()

        def row_body(r, rcarry):
            for g in range(D // LANES):
                s = pl.ds(g * LANES, LANES)
                acc_v[r, s] = acc_v[r, s] + b1_v[r, s] + b2_v[r, s]
            return rcarry

        lax.fori_loop(0, C, row_body, 0, unroll=False)
        pltpu.sync_copy(acc_v, out_hbm.at[pl.ds(off, C)])
        return carry

    lax.fori_loop(0, N_CHUNKS, chunk_body, 0, unroll=False)


@jax.jit
def _temporal_embedding(year_embed, month_embed, pos_embed, x_flat):
    run = pl.kernel(
        _body,
        out_type=jax.ShapeDtypeStruct((N, D), jnp.float32),
        mesh=plsc.VectorSubcoreMesh(core_axis_name="c", subcore_axis_name="s"),
        scratch_types=[
            pltpu.VMEM((C * 3,), jnp.int32),
            pltpu.VMEM((C,), jnp.int32),
            pltpu.VMEM((C,), jnp.int32),
            pltpu.VMEM((C,), jnp.int32),
            pltpu.VMEM((C, D), jnp.float32),
            pltpu.VMEM((C, D), jnp.float32),
            pltpu.VMEM((C, D), jnp.float32),
            pltpu.SemaphoreType.DMA,
        ],
        compiler_params=pltpu.CompilerParams(use_tc_tiling_on_sc=False,
                                             needs_layout_passes=False),
    )
    return run(year_embed, month_embed, pos_embed, x_flat)


def kernel(x, absolute_position_embed, year_embed, month_embed):
    x_flat = x.astype(jnp.int32).reshape(N * 3)
    out = _temporal_embedding(year_embed, month_embed, absolute_position_embed,
                              x_flat)
    return out.reshape(B, L, D)


# speedup vs baseline: 3.7296x; 3.7296x over previous
import jax
import jax.numpy as jnp
from jax import lax
from jax.experimental import pallas as pl
from jax.experimental.pallas import tpu as pltpu
from jax.experimental.pallas import tpu_sc as plsc

B = 4096
L = 200
D = 64
N = B * L
NC = 2
NS = 16
NW = NC * NS
PER_W = N // NW
C = 256
N_CHUNKS = PER_W // C
LANES = 16


def _body(year_hbm, month_hbm, pos_hbm, iy_hbm, im_hbm, ip_hbm, out_hbm,
          iy0, im0, ip0, g00, g10, g20,
          iy1, im1, ip1, g01, g11, g21,
          gsem0, gsem1, osem0, osem1):
    cid = lax.axis_index("c")
    sid = lax.axis_index("s")
    wid = sid * NC + cid
    base = wid * PER_W

    iy_v = (iy0, iy1)
    im_v = (im0, im1)
    ip_v = (ip0, ip1)
    g0 = (g00, g01)
    g1 = (g10, g11)
    g2 = (g20, g21)
    gsem = (gsem0, gsem1)
    osem = (osem0, osem1)

    def stage(k, s):
        off = pl.multiple_of(base + k * C, C)
        pltpu.sync_copy(iy_hbm.at[pl.ds(off, C)], iy_v[s])
        pltpu.sync_copy(im_hbm.at[pl.ds(off, C)], im_v[s])
        pltpu.sync_copy(ip_hbm.at[pl.ds(off, C)], ip_v[s])
        pltpu.async_copy(year_hbm.at[iy_v[s]], g0[s], gsem[s])
        pltpu.async_copy(month_hbm.at[im_v[s]], g1[s], gsem[s])
        pltpu.async_copy(pos_hbm.at[ip_v[s]], g2[s], gsem[s])

    def wait_gathers(s):
        pltpu.make_async_copy(year_hbm.at[iy_v[s]], g0[s], gsem[s]).wait()
        pltpu.make_async_copy(month_hbm.at[im_v[s]], g1[s], gsem[s]).wait()
        pltpu.make_async_copy(pos_hbm.at[ip_v[s]], g2[s], gsem[s]).wait()

    def compute(s):
        def row_body(r, rcarry):
            for g in range(D // LANES):
                sl = pl.ds(g * LANES, LANES)
                plsc.addupdate(g0[s].at[r, sl], g1[s][r, sl] + g2[s][r, sl])
            return rcarry

        lax.fori_loop(0, C, row_body, 0, unroll=False)

    def fire_out(k, s):
        off = pl.multiple_of(base + k * C, C)
        pltpu.async_copy(g0[s], out_hbm.at[pl.ds(off, C)], osem[s])

    def wait_out(s):
        pltpu.make_async_copy(g0[s], out_hbm.at[pl.ds(base, C)], osem[s]).wait()

    stage(0, 0)
    stage(1, 1)

    def pair_body(i2, carry):
        k0 = 2 * i2
        k1 = k0 + 1
        wait_gathers(0)
        compute(0)
        fire_out(k0, 0)
        wait_gathers(1)
        compute(1)
        fire_out(k1, 1)
        wait_out(0)
        stage(k0 + 2, 0)
        wait_out(1)
        stage(k1 + 2, 1)
        return carry

    lax.fori_loop(0, N_CHUNKS // 2 - 1, pair_body, 0, unroll=False)

    wait_gathers(0)
    compute(0)
    fire_out(N_CHUNKS - 2, 0)
    wait_gathers(1)
    compute(1)
    fire_out(N_CHUNKS - 1, 1)
    wait_out(0)
    wait_out(1)


@jax.jit
def _temporal_embedding(year_embed, month_embed, pos_embed, iy, im, ip):
    run = pl.kernel(
        _body,
        out_type=jax.ShapeDtypeStruct((N, D), jnp.float32),
        mesh=plsc.VectorSubcoreMesh(core_axis_name="c", subcore_axis_name="s"),
        scratch_types=[
            pltpu.VMEM((C,), jnp.int32),
            pltpu.VMEM((C,), jnp.int32),
            pltpu.VMEM((C,), jnp.int32),
            pltpu.VMEM((C, D), jnp.float32),
            pltpu.VMEM((C, D), jnp.float32),
            pltpu.VMEM((C, D), jnp.float32),
            pltpu.VMEM((C,), jnp.int32),
            pltpu.VMEM((C,), jnp.int32),
            pltpu.VMEM((C,), jnp.int32),
            pltpu.VMEM((C, D), jnp.float32),
            pltpu.VMEM((C, D), jnp.float32),
            pltpu.VMEM((C, D), jnp.float32),
            pltpu.SemaphoreType.DMA,
            pltpu.SemaphoreType.DMA,
            pltpu.SemaphoreType.DMA,
            pltpu.SemaphoreType.DMA,
        ],
        compiler_params=pltpu.CompilerParams(use_tc_tiling_on_sc=False),
    )
    return run(year_embed, month_embed, pos_embed, iy, im, ip)


def kernel(x, absolute_position_embed, year_embed, month_embed):
    idx = x.astype(jnp.int32).reshape(N, 3)
    iy = idx[:, 0]
    im = idx[:, 1]
    ip = idx[:, 2]
    out = _temporal_embedding(year_embed, month_embed, absolute_position_embed,
                              iy, im, ip)
    return out.reshape(B, L, D)
